# Initial kernel scaffold; baseline (speedup 1.0000x reference)
import jax
import jax.numpy as jnp
from jax.experimental import pallas as pl


def kernel(x, encoding):
    raise NotImplementedError("write your pallas kernel here")


# speedup vs baseline: 2.2949x; 2.2949x over previous
import functools

import jax
import jax.numpy as jnp
from jax import lax
from jax.experimental import pallas as pl
from jax.experimental.pallas import tpu as pltpu
from jax.experimental.pallas import tpu_sc as plsc

NUM_CORES = 2
NUM_SUBCORES = 16
NW = NUM_CORES * NUM_SUBCORES


@functools.partial(jax.jit, static_argnames=("n", "d", "c"))
def _gather_sc(idx, encoding, n, d, c):
    b_per_w = n // NW
    n_chunks = b_per_w // c
    mesh = plsc.VectorSubcoreMesh(core_axis_name="c", subcore_axis_name="s")

    @functools.partial(
        pl.kernel,
        mesh=mesh,
        out_type=jax.ShapeDtypeStruct((n, d), jnp.float32),
        scratch_types=[
            pltpu.VMEM((n_chunks, c), jnp.int32),
            pltpu.VMEM((c, d), jnp.float32),
            pltpu.VMEM((c, d), jnp.float32),
            pltpu.SemaphoreType.DMA,
            pltpu.SemaphoreType.DMA,
        ],
    )
    def k(enc_hbm, idx_hbm, out_hbm, idx_v, buf0, buf1, gsem0, gsem1):
        wid = lax.axis_index("s") * NUM_CORES + lax.axis_index("c")
        base = wid * b_per_w

        pltpu.sync_copy(idx_hbm.at[wid], idx_v)

        bufs = (buf0, buf1)
        sems = (gsem0, gsem1)

        def gather_start(j, slot):
            pltpu.async_copy(enc_hbm.at[idx_v.at[j]], bufs[slot], sems[slot])

        def gather_wait(j, slot):
            pltpu.make_async_copy(
                enc_hbm.at[idx_v.at[j]], bufs[slot], sems[slot]
            ).wait()

        def write_sync(j, slot):
            pltpu.sync_copy(bufs[slot], out_hbm.at[pl.ds(base + j * c, c)])

        gather_start(0, 0)

        def body(j0, carry):
            j = 2 * j0
            gather_wait(j, 0)
            gather_start(j + 1, 1)
            write_sync(j, 0)
            gather_wait(j + 1, 1)
            gather_start(j + 2, 0)
            write_sync(j + 1, 1)
            return carry

        lax.fori_loop(0, n_chunks // 2 - 1, body, 0)

        j = n_chunks - 2
        gather_wait(j, 0)
        gather_start(j + 1, 1)
        write_sync(j, 0)
        gather_wait(j + 1, 1)
        write_sync(j + 1, 1)

    return k(encoding, idx)


def kernel(x, encoding):
    b, s = x.shape
    v, d = encoding.shape
    n = b * s
    c = 32
    idx = x.reshape(NW, (n // NW) // c, c).astype(jnp.int32)
    out = _gather_sc(idx, encoding, n, d, c)
    return out.reshape(b, s, d)


# speedup vs baseline: 2.3898x; 1.0414x over previous
import functools

import jax
import jax.numpy as jnp
from jax import lax
from jax.experimental import pallas as pl
from jax.experimental.pallas import tpu as pltpu
from jax.experimental.pallas import tpu_sc as plsc

NUM_CORES = 2
NUM_SUBCORES = 16
NW = NUM_CORES * NUM_SUBCORES


@functools.partial(jax.jit, static_argnames=("n", "d", "c"))
def _gather_sc(idx, encoding, n, d, c):
    b_per_w = n // NW
    n_chunks = b_per_w // c
    mesh = plsc.VectorSubcoreMesh(core_axis_name="c", subcore_axis_name="s")

    @functools.partial(
        pl.kernel,
        mesh=mesh,
        out_type=jax.ShapeDtypeStruct((n, d), jnp.float32),
        scratch_types=[
            pltpu.VMEM((n_chunks, c), jnp.int32),
            pltpu.VMEM((c, d), jnp.float32),
            pltpu.VMEM((c, d), jnp.float32),
            pltpu.VMEM((c, d), jnp.float32),
            pltpu.VMEM((c, d), jnp.float32),
            pltpu.SemaphoreType.DMA,
            pltpu.SemaphoreType.DMA,
            pltpu.SemaphoreType.DMA,
            pltpu.SemaphoreType.DMA,
            pltpu.SemaphoreType.DMA,
            pltpu.SemaphoreType.DMA,
            pltpu.SemaphoreType.DMA,
            pltpu.SemaphoreType.DMA,
        ],
    )
    def k(enc_hbm, idx_hbm, out_hbm, idx_v, b0, b1, b2, b3,
          g0, g1, g2, g3, w0, w1, w2, w3):
        wid = lax.axis_index("s") * NUM_CORES + lax.axis_index("c")
        base = wid * b_per_w

        pltpu.sync_copy(idx_hbm.at[wid], idx_v)

        bufs = (b0, b1, b2, b3)
        gsems = (g0, g1, g2, g3)
        wsems = (w0, w1, w2, w3)

        def gather_start(j, slot):
            pltpu.async_copy(enc_hbm.at[idx_v.at[j]], bufs[slot], gsems[slot])

        def gather_wait(j, slot):
            pltpu.make_async_copy(
                enc_hbm.at[idx_v.at[j]], bufs[slot], gsems[slot]
            ).wait()

        def write_start(j, slot):
            pltpu.async_copy(
                bufs[slot], out_hbm.at[pl.ds(base + j * c, c)], wsems[slot]
            )

        def write_wait(j, slot):
            pltpu.make_async_copy(
                bufs[slot], out_hbm.at[pl.ds(base + j * c, c)], wsems[slot]
            ).wait()

        gather_start(0, 0)
        gather_start(1, 1)
        gather_wait(0, 0)
        write_start(0, 0)
        gather_start(2, 2)
        gather_wait(1, 1)
        write_start(1, 1)
        gather_start(3, 3)

        def body(g, carry):
            j0 = 2 + 4 * g
            for i in range(4):
                j = j0 + i
                slot = (2 + i) % 4
                gather_wait(j, slot)
                write_start(j, slot)
                write_wait(j - 2, (slot + 2) % 4)
                gather_start(j + 2, (slot + 2) % 4)
            return carry

        lax.fori_loop(0, (n_chunks - 4) // 4, body, 0)

        j = n_chunks - 2
        gather_wait(j, j % 4)
        write_start(j, j % 4)
        write_wait(j - 2, (j + 2) % 4)
        gather_wait(j + 1, (j + 1) % 4)
        write_start(j + 1, (j + 1) % 4)
        write_wait(j - 1, (j + 3) % 4)
        write_wait(j, j % 4)
        write_wait(j + 1, (j + 1) % 4)

    return k(encoding, idx)


def kernel(x, encoding):
    b, s = x.shape
    v, d = encoding.shape
    n = b * s
    c = 16
    idx = x.reshape(NW, (n // NW) // c, c).astype(jnp.int32)
    out = _gather_sc(idx, encoding, n, d, c)
    return out.reshape(b, s, d)
